# baseline (speedup 1.0000x reference)
import functools

import jax
import jax.numpy as jnp
from jax import lax
from jax.experimental import pallas as pl
from jax.experimental.pallas import tpu as pltpu
from jax.experimental.pallas import tpu_sc as plsc

NC = 2
NS = 16
L = 16

_SC_PARAMS = dict(
    compiler_params=pltpu.CompilerParams(
        needs_layout_passes=False, use_tc_tiling_on_sc=False))


def _geom(N, E, R):
    NH = N // NC
    NQ = NH // 2
    EPT = E // NS
    GK = 128
    NB = 2 * R
    CAP = EPT + NB * GK + L
    return NH, NQ, EPT, GK, NB, CAP


def _make_buckets(N, E, R):
    NH, NQ, EPT, GK, NB, CAP = _geom(N, E, R)
    SUB = 4000

    mesh = plsc.VectorSubcoreMesh(
        core_axis_name="c", subcore_axis_name="s",
        num_cores=NC, num_subcores=NS)

    @functools.partial(
        pl.kernel,
        out_type=(jax.ShapeDtypeStruct((NC, NS, 2, 1, CAP), jnp.int32),
                  jax.ShapeDtypeStruct((NC, NS, 1, L), jnp.int32)),
        mesh=mesh,
        scratch_types=[
            pltpu.VMEM((1, CAP), jnp.int32),
            pltpu.VMEM((1, CAP), jnp.int32),
            pltpu.VMEM((SUB,), jnp.int32),
            pltpu.VMEM((SUB,), jnp.int32),
            pltpu.VMEM((SUB,), jnp.int32),
            pltpu.VMEM((1, L), jnp.int32),
        ],
        **_SC_PARAMS,
    )
    def bucket(esrc, edst, etype, bko, bcnt,
               bsrc, bdst, tbuf, sbuf, dbuf, cbuf):
        c = lax.axis_index("c")
        s = lax.axis_index("s")
        ebase = s * EPT
        zvec_i = jnp.zeros((L,), jnp.int32)
        nvec_i = jnp.full((L,), NQ, jnp.int32)
        dlo = c * NH

        def count_sub(k, cnts):
            eb = ebase + k * SUB
            pltpu.sync_copy(etype.at[pl.ds(eb, SUB)], tbuf)
            pltpu.sync_copy(edst.at[pl.ds(eb, SUB)], dbuf)

            def count_vec(v, cnts):
                vb = pl.multiple_of(v * L, L)
                t = tbuf[pl.ds(vb, L)]
                d = dbuf[pl.ds(vb, L)]
                inh = (d >= dlo) & (d < dlo + NH)
                lower = d < dlo + NQ
                new = []
                for j in range(R):
                    m = (t == j) & inh
                    new.append(cnts[2 * j] + (m & lower).astype(jnp.int32))
                    new.append(cnts[2 * j + 1] +
                               (m & ~lower).astype(jnp.int32))
                return tuple(new)

            return lax.fori_loop(0, SUB // L, count_vec, cnts)

        cvecs = lax.fori_loop(0, EPT // SUB, count_sub,
                              tuple(jnp.zeros((L,), jnp.int32)
                                    for _ in range(NB)))
        cnts = tuple(jnp.sum(cvecs[i]) for i in range(NB))

        pads = tuple(((cnts[i] + GK - 1) // GK) * GK for i in range(NB))
        bases = []
        b = jnp.int32(0)
        for i in range(NB):
            bases.append(b)
            b = b + pads[i]
        bases = tuple(bases)

        def fill(i, _):
            ib = pl.multiple_of(i * L, L)
            bsrc[0, pl.ds(ib, L)] = zvec_i
            bdst[0, pl.ds(ib, L)] = nvec_i
            return 0
        lax.fori_loop(0, CAP // L, fill, 0)

        def comp_sub(k, offs):
            eb = ebase + k * SUB
            pltpu.sync_copy(etype.at[pl.ds(eb, SUB)], tbuf)
            pltpu.sync_copy(esrc.at[pl.ds(eb, SUB)], sbuf)
            pltpu.sync_copy(edst.at[pl.ds(eb, SUB)], dbuf)

            def comp_vec(v, offs):
                vb = pl.multiple_of(v * L, L)
                t = tbuf[pl.ds(vb, L)]
                sv = sbuf[pl.ds(vb, L)]
                d = dbuf[pl.ds(vb, L)]
                zl = jnp.zeros((L,), jnp.int32)
                inh = (d >= dlo) & (d < dlo + NH)
                dr = d - dlo
                lower = dr < NQ
                drq = jnp.where(lower, dr, dr - NQ)
                new = []
                for j in range(R):
                    m = (t == j) & inh
                    mf = (m & lower).astype(jnp.int32)
                    mb = (m & ~lower).astype(jnp.int32)
                    csf = plsc.cumsum(mf)
                    csb = plsc.cumsum(mb)
                    gpos = jnp.where(lower,
                                     offs[2 * j] + csf - 1,
                                     offs[2 * j + 1] - csb)
                    plsc.store_scatter(bsrc, [zl, gpos], sv, mask=m)
                    plsc.store_scatter(bdst, [zl, gpos], drq, mask=m)
                    new.append(offs[2 * j] + jnp.sum(mf))
                    new.append(offs[2 * j + 1] - jnp.sum(mb))
                return tuple(new)

            return lax.fori_loop(0, SUB // L, comp_vec, offs)

        offs0 = tuple(bases[i] if i % 2 == 0 else bases[i] + pads[i]
                      for i in range(NB))
        lax.fori_loop(0, EPT // SUB, comp_sub, offs0)

        pltpu.sync_copy(bsrc, bko.at[c, s, 0])
        pltpu.sync_copy(bdst, bko.at[c, s, 1])
        iot = lax.iota(jnp.int32, L)
        cv = jnp.zeros((L,), jnp.int32)
        for i in range(NB):
            cv = jnp.where(iot == i, cnts[i], cv)
        cbuf[0, pl.ds(0, L)] = cv
        pltpu.sync_copy(cbuf, bcnt.at[c, s])

    return bucket


def _make_agg(N, E, D, R, DE):
    NH, NQ, EPT, GK, NB, CAP = _geom(N, E, R)
    PAD = 8
    NP = NQ + PAD
    NBUF = 3
    ZR = NP // NS // 8 * 8
    ZTAIL = NP - NS * ZR
    DR = NQ // NS // 8 * 8
    DTAIL = NQ - NS * DR

    mesh = plsc.VectorSubcoreMesh(
        core_axis_name="c", subcore_axis_name="s",
        num_cores=NC, num_subcores=NS)

    @functools.partial(
        pl.kernel,
        out_type=jax.ShapeDtypeStruct((R, N, DE), jnp.float32),
        mesh=mesh,
        scratch_types=[
            pltpu.VMEM((1, CAP), jnp.int32),
            pltpu.VMEM((1, CAP), jnp.int32),
            pltpu.VMEM((1, L), jnp.int32),
            pltpu.VMEM((NBUF, GK, DE), jnp.float32),
            pltpu.VMEM_SHARED((NP, DE), jnp.float32),
            pltpu.SemaphoreType.DMA,
            pltpu.SemaphoreType.DMA,
            pltpu.SemaphoreType.DMA,
        ],
        **_SC_PARAMS,
    )
    def agg(xe, zin, bko, bcnt, out,
            bsrc, bdst, cbuf, rows, acc, sem0, sem1, sem2):
        c = lax.axis_index("c")
        s = lax.axis_index("s")
        sems = (sem0, sem1, sem2)

        pltpu.sync_copy(bko.at[c, s, 0], bsrc)
        pltpu.sync_copy(bko.at[c, s, 1], bdst)
        pltpu.sync_copy(bcnt.at[c, s], cbuf)
        cv = cbuf[0, pl.ds(0, L)]
        iot = lax.iota(jnp.int32, L)
        pads_v = (cv + GK - 1) // GK * GK
        bases_v = plsc.cumsum(pads_v) - pads_v
        dlo = c * NH

        def region(ireg, _):
            nch = jnp.sum(jnp.where(iot == ireg, pads_v, 0)) // GK
            rbase = jnp.sum(jnp.where(iot == ireg, bases_v, 0))
            j = ireg // 2
            h = ireg % 2

            for q in range(ZR // GK + (1 if ZR % GK else 0)):
                w = min(GK, ZR - q * GK)
                zo = pl.multiple_of(s * ZR + q * GK, 8)
                pltpu.sync_copy(zin.at[pl.ds(0, w)], acc.at[pl.ds(zo, w)])

            @pl.when(s == NS - 1)
            def _zero_tail():
                pltpu.sync_copy(zin.at[pl.ds(0, ZTAIL)],
                                acc.at[pl.ds(NS * ZR, ZTAIL)])

            plsc.subcore_barrier()

            nsh = nch // 2

            def gidx(g):
                off = pl.multiple_of(rbase + g * GK, GK)
                return bsrc.at[0, pl.ds(off, GK)]

            def didx(g):
                off = pl.multiple_of(rbase + g * GK, GK)
                return bdst.at[0, pl.ds(off, GK)]

            def rissue(g):
                boff = rbase + g * GK

                def ibody(v, _):
                    vb = pl.multiple_of(boff + v * L, 8)
                    iv = bsrc[0, pl.ds(vb, L)]
                    for r in range(L):
                        si = iv[r]
                        pltpu.async_copy(xe.at[pl.ds(si, 1)],
                                         rows.at[2, pl.ds(v * L + r, 1)],
                                         sems[2])
                    return 0

                lax.fori_loop(0, GK // L, ibody, 0)

            def rdrain():
                def dbody(r, _):
                    pltpu.make_async_copy(xe.at[pl.ds(0, 1)],
                                          rows.at[2, pl.ds(0, 1)],
                                          sems[2]).wait()
                    return 0

                lax.fori_loop(0, GK, dbody, 0)

            for bi in range(2):
                @pl.when(bi < nsh)
                def _prime(bi=bi):
                    pltpu.async_copy(xe.at[gidx(bi)], rows.at[bi], sems[bi])

            @pl.when(nsh < nch)
            def _prime_row():
                rissue(nsh)

            def outer(q, _):
                for bi in range(2):
                    g = q * 2 + bi

                    @pl.when(g < nsh)
                    def _step(bi=bi, g=g):
                        pltpu.make_async_copy(xe.at[gidx(g)], rows.at[bi],
                                              sems[bi]).wait()
                        pltpu.sync_copy(rows.at[bi], acc.at[didx(g)],
                                        add=True)

                        @pl.when(g + 2 < nsh)
                        def _next():
                            pltpu.async_copy(xe.at[gidx(g + 2)],
                                             rows.at[bi], sems[bi])

                gr = nsh + q

                @pl.when(gr < nch)
                def _rstep():
                    rdrain()
                    pltpu.sync_copy(rows.at[2], acc.at[didx(gr)], add=True)

                    @pl.when(gr + 1 < nch)
                    def _rnext():
                        rissue(gr + 1)
                return 0

            nloop = jnp.maximum((nsh + 1) // 2, nch - nsh)
            lax.fori_loop(0, nloop, outer, 0)
            plsc.subcore_barrier()
            obase = dlo + h * NQ
            pltpu.sync_copy(acc.at[pl.ds(s * DR, DR)],
                            out.at[j, pl.ds(obase + s * DR, DR)])

            @pl.when(s == NS - 1)
            def _dump_tail():
                pltpu.sync_copy(acc.at[pl.ds(NS * DR, DTAIL)],
                                out.at[j, pl.ds(obase + NS * DR, DTAIL)])

            plsc.subcore_barrier()
            return 0

        lax.fori_loop(0, NB, region, 0)

    return agg


def _make_dense(N, D, R, DE, BLK, emit_ext):
    def body(acc_ref, x_ref, w_ref, root_ref, b_ref, o_ref):
        x = x_ref[:, :D]
        y = jnp.dot(x, root_ref[...], preferred_element_type=jnp.float32)
        y = y + b_ref[...]
        for r in range(R):
            a = acc_ref[r, :, :D]
            cnt = acc_ref[r, :, D:D + 1]
            inv = 1.0 / jnp.maximum(cnt, 1.0)
            y = y + jnp.dot(a * inv, w_ref[r],
                            preferred_element_type=jnp.float32)
        if emit_ext:
            y = jnp.maximum(y, 0.0)
            col = lax.broadcasted_iota(jnp.int32, (BLK, DE - D), 1)
            tail = jnp.where(col == 0, 1.0, 0.0).astype(jnp.float32)
            o_ref[...] = jnp.concatenate([y, tail], axis=1)
        else:
            o_ref[...] = y

    out_w = DE if emit_ext else D
    return pl.pallas_call(
        body,
        grid=(N // BLK,),
        in_specs=[
            pl.BlockSpec((R, BLK, DE), lambda i: (0, i, 0)),
            pl.BlockSpec((BLK, DE), lambda i: (i, 0)),
            pl.BlockSpec((R, D, D), lambda i: (0, 0, 0)),
            pl.BlockSpec((D, D), lambda i: (0, 0)),
            pl.BlockSpec((D,), lambda i: (0,)),
        ],
        out_specs=pl.BlockSpec((BLK, out_w), lambda i: (i, 0)),
        out_shape=jax.ShapeDtypeStruct((N, out_w), jnp.float32),
    )


def kernel(edge_index, edge_type, emb, W1, root1, b1, W2, root2, b2):
    N, D = emb.shape
    E = edge_index.shape[1]
    R = W1.shape[0]
    DE = 144

    esrc = edge_index[0]
    edst = edge_index[1]

    bucket = _make_buckets(N, E, R)
    agg = _make_agg(N, E, D, R, DE)
    dense1 = _make_dense(N, D, R, DE, 1000, True)
    dense2 = _make_dense(N, D, R, DE, 1000, False)

    pad = jnp.zeros((N, DE - D - 1), jnp.float32)
    ones = jnp.ones((N, 1), jnp.float32)
    xe0 = jnp.concatenate([emb, ones, pad], axis=1)

    zin = jnp.zeros((128, DE), jnp.float32)
    bko, bcnt = bucket(esrc, edst, edge_type)
    acc1 = agg(xe0, zin, bko, bcnt)
    xe1 = dense1(acc1, xe0, W1, root1, b1)
    acc2 = agg(xe1, zin, bko, bcnt)
    y = dense2(acc2, xe1, W2, root2, b2)
    return y


# speedup vs baseline: 2.2371x; 2.2371x over previous
import functools

import jax
import jax.numpy as jnp
from jax import lax
from jax.experimental import pallas as pl
from jax.experimental.pallas import tpu as pltpu
from jax.experimental.pallas import tpu_sc as plsc

NC = 2
NS = 16
L = 16

_SC_PARAMS = dict(
    compiler_params=pltpu.CompilerParams(
        needs_layout_passes=False, use_tc_tiling_on_sc=False))


def _geom(N, E, R):
    NH = N // NC
    NQ = NH // 2
    EPT = E // NS
    GK = 128
    NB = 2 * R
    CAP = EPT + NB * GK + L
    return NH, NQ, EPT, GK, NB, CAP


def _make_buckets(N, E, R):
    NH, NQ, EPT, GK, NB, CAP = _geom(N, E, R)
    SUB = 4000

    mesh = plsc.VectorSubcoreMesh(
        core_axis_name="c", subcore_axis_name="s",
        num_cores=NC, num_subcores=NS)

    @functools.partial(
        pl.kernel,
        out_type=(jax.ShapeDtypeStruct((NC, NS, 2, 1, CAP), jnp.int32),
                  jax.ShapeDtypeStruct((NC, NS, 1, L), jnp.int32)),
        mesh=mesh,
        scratch_types=[
            pltpu.VMEM((1, CAP), jnp.int32),
            pltpu.VMEM((1, CAP), jnp.int32),
            pltpu.VMEM((SUB,), jnp.int32),
            pltpu.VMEM((SUB,), jnp.int32),
            pltpu.VMEM((SUB,), jnp.int32),
            pltpu.VMEM((1, L), jnp.int32),
        ],
        **_SC_PARAMS,
    )
    def bucket(esrc, edst, etype, bko, bcnt,
               bsrc, bdst, tbuf, sbuf, dbuf, cbuf):
        c = lax.axis_index("c")
        s = lax.axis_index("s")
        ebase = s * EPT
        zvec_i = jnp.zeros((L,), jnp.int32)
        nvec_i = jnp.full((L,), NQ, jnp.int32)
        dlo = c * NH

        def count_sub(k, cnts):
            eb = ebase + k * SUB
            pltpu.sync_copy(etype.at[pl.ds(eb, SUB)], tbuf)
            pltpu.sync_copy(edst.at[pl.ds(eb, SUB)], dbuf)

            def count_vec(v, cnts):
                vb = pl.multiple_of(v * L, L)
                t = tbuf[pl.ds(vb, L)]
                d = dbuf[pl.ds(vb, L)]
                inh = (d >= dlo) & (d < dlo + NH)
                lower = d < dlo + NQ
                new = []
                for j in range(R):
                    m = (t == j) & inh
                    new.append(cnts[2 * j] + (m & lower).astype(jnp.int32))
                    new.append(cnts[2 * j + 1] +
                               (m & ~lower).astype(jnp.int32))
                return tuple(new)

            return lax.fori_loop(0, SUB // L, count_vec, cnts)

        cvecs = lax.fori_loop(0, EPT // SUB, count_sub,
                              tuple(jnp.zeros((L,), jnp.int32)
                                    for _ in range(NB)))
        cnts = tuple(jnp.sum(cvecs[i]) for i in range(NB))

        pads = tuple(((cnts[i] + GK - 1) // GK) * GK for i in range(NB))
        bases = []
        b = jnp.int32(0)
        for i in range(NB):
            bases.append(b)
            b = b + pads[i]
        bases = tuple(bases)

        def fill(i, _):
            ib = pl.multiple_of(i * L, L)
            bsrc[0, pl.ds(ib, L)] = zvec_i
            bdst[0, pl.ds(ib, L)] = nvec_i
            return 0
        lax.fori_loop(0, CAP // L, fill, 0)

        def comp_sub(k, offs):
            eb = ebase + k * SUB
            pltpu.sync_copy(etype.at[pl.ds(eb, SUB)], tbuf)
            pltpu.sync_copy(esrc.at[pl.ds(eb, SUB)], sbuf)
            pltpu.sync_copy(edst.at[pl.ds(eb, SUB)], dbuf)

            def comp_vec(v, offs):
                vb = pl.multiple_of(v * L, L)
                t = tbuf[pl.ds(vb, L)]
                sv = sbuf[pl.ds(vb, L)]
                d = dbuf[pl.ds(vb, L)]
                zl = jnp.zeros((L,), jnp.int32)
                inh = (d >= dlo) & (d < dlo + NH)
                dr = d - dlo
                lower = dr < NQ
                drq = jnp.where(lower, dr, dr - NQ)
                new = []
                for j in range(R):
                    m = (t == j) & inh
                    mf = (m & lower).astype(jnp.int32)
                    mb = (m & ~lower).astype(jnp.int32)
                    csf = plsc.cumsum(mf)
                    csb = plsc.cumsum(mb)
                    gpos = jnp.where(lower,
                                     offs[2 * j] + csf - 1,
                                     offs[2 * j + 1] - csb)
                    plsc.store_scatter(bsrc, [zl, gpos], sv, mask=m)
                    plsc.store_scatter(bdst, [zl, gpos], drq, mask=m)
                    new.append(offs[2 * j] + jnp.sum(mf))
                    new.append(offs[2 * j + 1] - jnp.sum(mb))
                return tuple(new)

            return lax.fori_loop(0, SUB // L, comp_vec, offs)

        offs0 = tuple(bases[i] if i % 2 == 0 else bases[i] + pads[i]
                      for i in range(NB))
        lax.fori_loop(0, EPT // SUB, comp_sub, offs0)

        pltpu.sync_copy(bsrc, bko.at[c, s, 0])
        pltpu.sync_copy(bdst, bko.at[c, s, 1])
        iot = lax.iota(jnp.int32, L)
        cv = jnp.zeros((L,), jnp.int32)
        for i in range(NB):
            cv = jnp.where(iot == i, cnts[i], cv)
        cbuf[0, pl.ds(0, L)] = cv
        pltpu.sync_copy(cbuf, bcnt.at[c, s])

    return bucket


def _make_agg(N, E, D, R, DE):
    NH, NQ, EPT, GK, NB, CAP = _geom(N, E, R)
    PAD = 8
    NP = NQ + PAD
    NBUF = 3
    ZR = NP // NS // 8 * 8
    ZTAIL = NP - NS * ZR
    DR = NQ // NS // 8 * 8
    DTAIL = NQ - NS * DR

    mesh = plsc.VectorSubcoreMesh(
        core_axis_name="c", subcore_axis_name="s",
        num_cores=NC, num_subcores=NS)

    @functools.partial(
        pl.kernel,
        out_type=jax.ShapeDtypeStruct((R, N, DE), jnp.float32),
        mesh=mesh,
        scratch_types=[
            pltpu.VMEM((1, CAP), jnp.int32),
            pltpu.VMEM((1, CAP), jnp.int32),
            pltpu.VMEM((1, L), jnp.int32),
            pltpu.VMEM((NBUF, GK, DE), jnp.float32),
            pltpu.VMEM_SHARED((NP, DE), jnp.float32),
            pltpu.SemaphoreType.DMA,
            pltpu.SemaphoreType.DMA,
            pltpu.SemaphoreType.DMA,
        ],
        **_SC_PARAMS,
    )
    def agg(xe, zin, bko, bcnt, out,
            bsrc, bdst, cbuf, rows, acc, sem0, sem1, sem2):
        c = lax.axis_index("c")
        s = lax.axis_index("s")
        sems = (sem0, sem1, sem2)

        pltpu.sync_copy(bko.at[c, s, 0], bsrc)
        pltpu.sync_copy(bko.at[c, s, 1], bdst)
        pltpu.sync_copy(bcnt.at[c, s], cbuf)
        cv = cbuf[0, pl.ds(0, L)]
        iot = lax.iota(jnp.int32, L)
        pads_v = (cv + GK - 1) // GK * GK
        bases_v = plsc.cumsum(pads_v) - pads_v
        dlo = c * NH

        def region(ireg, _):
            nch = jnp.sum(jnp.where(iot == ireg, pads_v, 0)) // GK
            rbase = jnp.sum(jnp.where(iot == ireg, bases_v, 0))
            j = ireg // 2
            h = ireg % 2

            for q in range(ZR // GK + (1 if ZR % GK else 0)):
                w = min(GK, ZR - q * GK)
                zo = pl.multiple_of(s * ZR + q * GK, 8)
                pltpu.sync_copy(zin.at[pl.ds(0, w)], acc.at[pl.ds(zo, w)])

            @pl.when(s == NS - 1)
            def _zero_tail():
                pltpu.sync_copy(zin.at[pl.ds(0, ZTAIL)],
                                acc.at[pl.ds(NS * ZR, ZTAIL)])

            plsc.subcore_barrier()

            cnt_r = jnp.sum(jnp.where(iot == ireg, cv, 0))
            nf = cnt_r // GK
            rem = cnt_r - nf * GK
            remv = (rem + L - 1) // L
            h2 = ireg % 2
            gs = jnp.where(h2 == 1, nch - nf, 0)
            pcg = jnp.where(h2 == 1, 0, nf)
            lstart = jnp.where(h2 == 1, GK - remv * L, 0)

            def gidx(g):
                off = pl.multiple_of(rbase + g * GK, GK)
                return bsrc.at[0, pl.ds(off, GK)]

            def didx(g):
                off = pl.multiple_of(rbase + g * GK, GK)
                return bdst.at[0, pl.ds(off, GK)]

            @pl.when(rem > 0)
            def _issue_tail():
                def ibody(v, _):
                    lane0 = lstart + v * L
                    vb = pl.multiple_of(rbase + pcg * GK + lane0, 8)
                    iv = bsrc[0, pl.ds(vb, L)]
                    for r in range(L):
                        si = iv[r]
                        pltpu.async_copy(xe.at[pl.ds(si, 1)],
                                         rows.at[2, pl.ds(lane0 + r, 1)],
                                         sems[2])
                    return 0

                lax.fori_loop(0, remv, ibody, 0)

            for bi in range(2):
                @pl.when(bi < nf)
                def _prime(bi=bi):
                    pltpu.async_copy(xe.at[gidx(gs + bi)], rows.at[bi],
                                     sems[bi])

            def outer(q, _):
                for bi in range(2):
                    k = q * 2 + bi
                    g = gs + k

                    @pl.when(k < nf)
                    def _step(bi=bi, g=g, k=k):
                        pltpu.make_async_copy(xe.at[gidx(g)], rows.at[bi],
                                              sems[bi]).wait()
                        pltpu.sync_copy(rows.at[bi], acc.at[didx(g)],
                                        add=True)

                        @pl.when(k + 2 < nf)
                        def _next():
                            pltpu.async_copy(xe.at[gidx(g + 2)],
                                             rows.at[bi], sems[bi])
                return 0

            lax.fori_loop(0, (nf + 1) // 2, outer, 0)

            @pl.when(rem > 0)
            def _drain_tail():
                def dbody(r, _):
                    pltpu.make_async_copy(xe.at[pl.ds(0, 1)],
                                          rows.at[2, pl.ds(0, 1)],
                                          sems[2]).wait()
                    return 0

                lax.fori_loop(0, remv * L, dbody, 0)
                pltpu.sync_copy(rows.at[2], acc.at[didx(pcg)], add=True)
            plsc.subcore_barrier()
            obase = dlo + h * NQ
            pltpu.sync_copy(acc.at[pl.ds(s * DR, DR)],
                            out.at[j, pl.ds(obase + s * DR, DR)])

            @pl.when(s == NS - 1)
            def _dump_tail():
                pltpu.sync_copy(acc.at[pl.ds(NS * DR, DTAIL)],
                                out.at[j, pl.ds(obase + NS * DR, DTAIL)])

            plsc.subcore_barrier()
            return 0

        lax.fori_loop(0, NB, region, 0)

    return agg


def _make_dense(N, D, R, DE, BLK, emit_ext):
    def body(acc_ref, x_ref, w_ref, root_ref, b_ref, o_ref):
        x = x_ref[:, :D]
        y = jnp.dot(x, root_ref[...], preferred_element_type=jnp.float32)
        y = y + b_ref[...]
        for r in range(R):
            a = acc_ref[r, :, :D]
            cnt = acc_ref[r, :, D:D + 1]
            inv = 1.0 / jnp.maximum(cnt, 1.0)
            y = y + jnp.dot(a * inv, w_ref[r],
                            preferred_element_type=jnp.float32)
        if emit_ext:
            y = jnp.maximum(y, 0.0)
            col = lax.broadcasted_iota(jnp.int32, (BLK, DE - D), 1)
            tail = jnp.where(col == 0, 1.0, 0.0).astype(jnp.float32)
            o_ref[...] = jnp.concatenate([y, tail], axis=1)
        else:
            o_ref[...] = y

    out_w = DE if emit_ext else D
    return pl.pallas_call(
        body,
        grid=(N // BLK,),
        in_specs=[
            pl.BlockSpec((R, BLK, DE), lambda i: (0, i, 0)),
            pl.BlockSpec((BLK, DE), lambda i: (i, 0)),
            pl.BlockSpec((R, D, D), lambda i: (0, 0, 0)),
            pl.BlockSpec((D, D), lambda i: (0, 0)),
            pl.BlockSpec((D,), lambda i: (0,)),
        ],
        out_specs=pl.BlockSpec((BLK, out_w), lambda i: (i, 0)),
        out_shape=jax.ShapeDtypeStruct((N, out_w), jnp.float32),
    )


def kernel(edge_index, edge_type, emb, W1, root1, b1, W2, root2, b2):
    N, D = emb.shape
    E = edge_index.shape[1]
    R = W1.shape[0]
    DE = 136

    esrc = edge_index[0]
    edst = edge_index[1]

    bucket = _make_buckets(N, E, R)
    agg = _make_agg(N, E, D, R, DE)
    dense1 = _make_dense(N, D, R, DE, 1000, True)
    dense2 = _make_dense(N, D, R, DE, 1000, False)

    pad = jnp.zeros((N, DE - D - 1), jnp.float32)
    ones = jnp.ones((N, 1), jnp.float32)
    xe0 = jnp.concatenate([emb, ones, pad], axis=1)

    zin = jnp.zeros((128, DE), jnp.float32)
    bko, bcnt = bucket(esrc, edst, edge_type)
    acc1 = agg(xe0, zin, bko, bcnt)
    xe1 = dense1(acc1, xe0, W1, root1, b1)
    acc2 = agg(xe1, zin, bko, bcnt)
    y = dense2(acc2, xe1, W2, root2, b2)
    return y


# speedup vs baseline: 2.2782x; 1.0184x over previous
import functools

import jax
import jax.numpy as jnp
from jax import lax
from jax.experimental import pallas as pl
from jax.experimental.pallas import tpu as pltpu
from jax.experimental.pallas import tpu_sc as plsc

NC = 2
NS = 16
L = 16

_SC_PARAMS = dict(
    compiler_params=pltpu.CompilerParams(
        needs_layout_passes=False, use_tc_tiling_on_sc=False))


def _geom(N, E, R):
    NH = N // NC
    NQ = NH // 2
    EPT = E // NS
    GK = 128
    NB = 2 * R
    CAP = EPT + NB * GK + L
    return NH, NQ, EPT, GK, NB, CAP


def _make_buckets(N, E, R):
    NH, NQ, EPT, GK, NB, CAP = _geom(N, E, R)
    SUB = 4000

    mesh = plsc.VectorSubcoreMesh(
        core_axis_name="c", subcore_axis_name="s",
        num_cores=NC, num_subcores=NS)

    @functools.partial(
        pl.kernel,
        out_type=(jax.ShapeDtypeStruct((NC, NS, 2, 1, CAP), jnp.int32),
                  jax.ShapeDtypeStruct((NC, NS, 1, L), jnp.int32)),
        mesh=mesh,
        scratch_types=[
            pltpu.VMEM((1, CAP), jnp.int32),
            pltpu.VMEM((1, CAP), jnp.int32),
            pltpu.VMEM((SUB,), jnp.int32),
            pltpu.VMEM((SUB,), jnp.int32),
            pltpu.VMEM((SUB,), jnp.int32),
            pltpu.VMEM((1, L), jnp.int32),
        ],
        **_SC_PARAMS,
    )
    def bucket(esrc, edst, etype, bko, bcnt,
               bsrc, bdst, tbuf, sbuf, dbuf, cbuf):
        c = lax.axis_index("c")
        s = lax.axis_index("s")
        ebase = s * EPT
        zvec_i = jnp.zeros((L,), jnp.int32)
        nvec_i = jnp.full((L,), NQ, jnp.int32)
        dlo = c * NH

        def count_sub(k, cnts):
            eb = ebase + k * SUB
            pltpu.sync_copy(etype.at[pl.ds(eb, SUB)], tbuf)
            pltpu.sync_copy(edst.at[pl.ds(eb, SUB)], dbuf)

            def count_vec(v, cnts):
                vb = pl.multiple_of(v * L, L)
                t = tbuf[pl.ds(vb, L)]
                d = dbuf[pl.ds(vb, L)]
                inh = (d >= dlo) & (d < dlo + NH)
                lower = d < dlo + NQ
                new = []
                for j in range(R):
                    m = (t == j) & inh
                    new.append(cnts[2 * j] + (m & lower).astype(jnp.int32))
                    new.append(cnts[2 * j + 1] +
                               (m & ~lower).astype(jnp.int32))
                return tuple(new)

            return lax.fori_loop(0, SUB // L, count_vec, cnts)

        cvecs = lax.fori_loop(0, EPT // SUB, count_sub,
                              tuple(jnp.zeros((L,), jnp.int32)
                                    for _ in range(NB)))
        cnts = tuple(jnp.sum(cvecs[i]) for i in range(NB))

        pads = tuple(((cnts[i] + GK - 1) // GK) * GK for i in range(NB))
        bases = []
        b = jnp.int32(0)
        for i in range(NB):
            bases.append(b)
            b = b + pads[i]
        bases = tuple(bases)

        def fill(i, _):
            ib = pl.multiple_of(i * L, L)
            bsrc[0, pl.ds(ib, L)] = zvec_i
            bdst[0, pl.ds(ib, L)] = nvec_i
            return 0
        lax.fori_loop(0, CAP // L, fill, 0)

        def comp_sub(k, offs):
            eb = ebase + k * SUB
            pltpu.sync_copy(etype.at[pl.ds(eb, SUB)], tbuf)
            pltpu.sync_copy(esrc.at[pl.ds(eb, SUB)], sbuf)
            pltpu.sync_copy(edst.at[pl.ds(eb, SUB)], dbuf)

            def comp_vec(v, offs):
                vb = pl.multiple_of(v * L, L)
                t = tbuf[pl.ds(vb, L)]
                sv = sbuf[pl.ds(vb, L)]
                d = dbuf[pl.ds(vb, L)]
                zl = jnp.zeros((L,), jnp.int32)
                inh = (d >= dlo) & (d < dlo + NH)
                dr = d - dlo
                lower = dr < NQ
                drq = jnp.where(lower, dr, dr - NQ)
                new = []
                for j in range(R):
                    m = (t == j) & inh
                    mf = (m & lower).astype(jnp.int32)
                    mb = (m & ~lower).astype(jnp.int32)
                    csf = plsc.cumsum(mf)
                    csb = plsc.cumsum(mb)
                    gpos = jnp.where(lower,
                                     offs[2 * j] + csf - 1,
                                     offs[2 * j + 1] - csb)
                    plsc.store_scatter(bsrc, [zl, gpos], sv, mask=m)
                    plsc.store_scatter(bdst, [zl, gpos], drq, mask=m)
                    new.append(offs[2 * j] + jnp.sum(mf))
                    new.append(offs[2 * j + 1] - jnp.sum(mb))
                return tuple(new)

            return lax.fori_loop(0, SUB // L, comp_vec, offs)

        offs0 = tuple(bases[i] if i % 2 == 0 else bases[i] + pads[i]
                      for i in range(NB))
        lax.fori_loop(0, EPT // SUB, comp_sub, offs0)

        pltpu.sync_copy(bsrc, bko.at[c, s, 0])
        pltpu.sync_copy(bdst, bko.at[c, s, 1])
        iot = lax.iota(jnp.int32, L)
        cv = jnp.zeros((L,), jnp.int32)
        for i in range(NB):
            cv = jnp.where(iot == i, cnts[i], cv)
        cbuf[0, pl.ds(0, L)] = cv
        pltpu.sync_copy(cbuf, bcnt.at[c, s])

    return bucket


def _make_agg(N, E, D, R, DE):
    NH, NQ, EPT, GK, NB, CAP = _geom(N, E, R)
    PAD = 8
    NP = NQ + PAD
    NBUF = 3
    ZR = NP // NS // 8 * 8
    ZTAIL = NP - NS * ZR
    DR = NQ // NS // 8 * 8
    DTAIL = NQ - NS * DR

    mesh = plsc.VectorSubcoreMesh(
        core_axis_name="c", subcore_axis_name="s",
        num_cores=NC, num_subcores=NS)

    @functools.partial(
        pl.kernel,
        out_type=jax.ShapeDtypeStruct((R, N, DE), jnp.float32),
        mesh=mesh,
        scratch_types=[
            pltpu.VMEM((1, CAP), jnp.int32),
            pltpu.VMEM((1, CAP), jnp.int32),
            pltpu.VMEM((1, L), jnp.int32),
            pltpu.VMEM((NBUF, GK, DE), jnp.float32),
            pltpu.VMEM_SHARED((NP, DE), jnp.float32),
            pltpu.SemaphoreType.DMA,
            pltpu.SemaphoreType.DMA,
            pltpu.SemaphoreType.DMA,
        ],
        **_SC_PARAMS,
    )
    def agg(xe, zin, bko, bcnt, out,
            bsrc, bdst, cbuf, rows, acc, sem0, sem1, sem2):
        c = lax.axis_index("c")
        s = lax.axis_index("s")
        sems = (sem0, sem1, sem2)

        pltpu.sync_copy(bko.at[c, s, 0], bsrc)
        pltpu.sync_copy(bko.at[c, s, 1], bdst)
        pltpu.sync_copy(bcnt.at[c, s], cbuf)
        cv = cbuf[0, pl.ds(0, L)]
        iot = lax.iota(jnp.int32, L)
        pads_v = (cv + GK - 1) // GK * GK
        bases_v = plsc.cumsum(pads_v) - pads_v
        dlo = c * NH

        def region(ireg, _):
            nch = jnp.sum(jnp.where(iot == ireg, pads_v, 0)) // GK
            rbase = jnp.sum(jnp.where(iot == ireg, bases_v, 0))
            j = ireg // 2
            h = ireg % 2

            @pl.when(ireg > 0)
            def _wait_prev_dump():
                pltpu.make_async_copy(acc.at[pl.ds(0, DR)],
                                      out.at[0, pl.ds(0, DR)],
                                      sems[0]).wait()

                @pl.when(s == NS - 1)
                def _wait_prev_tail():
                    pltpu.make_async_copy(acc.at[pl.ds(0, DTAIL)],
                                          out.at[0, pl.ds(0, DTAIL)],
                                          sems[0]).wait()

            for q in range(ZR // GK + (1 if ZR % GK else 0)):
                w = min(GK, ZR - q * GK)
                zo = pl.multiple_of(s * ZR + q * GK, 8)
                pltpu.sync_copy(zin.at[pl.ds(0, w)], acc.at[pl.ds(zo, w)])

            @pl.when(s == NS - 1)
            def _zero_tail():
                pltpu.sync_copy(zin.at[pl.ds(0, ZTAIL)],
                                acc.at[pl.ds(NS * ZR, ZTAIL)])

            plsc.subcore_barrier()

            cnt_r = jnp.sum(jnp.where(iot == ireg, cv, 0))
            nf = cnt_r // GK
            rem = cnt_r - nf * GK
            remv = (rem + L - 1) // L
            h2 = ireg % 2
            gs = jnp.where(h2 == 1, nch - nf, 0)
            pcg = jnp.where(h2 == 1, 0, nf)
            lstart = jnp.where(h2 == 1, GK - remv * L, 0)

            def gidx(g):
                off = pl.multiple_of(rbase + g * GK, GK)
                return bsrc.at[0, pl.ds(off, GK)]

            def didx(g):
                off = pl.multiple_of(rbase + g * GK, GK)
                return bdst.at[0, pl.ds(off, GK)]

            @pl.when(rem > 0)
            def _issue_tail():
                def ibody(v, _):
                    lane0 = lstart + v * L
                    vb = pl.multiple_of(rbase + pcg * GK + lane0, 8)
                    iv = bsrc[0, pl.ds(vb, L)]
                    for r in range(L):
                        si = iv[r]
                        pltpu.async_copy(xe.at[pl.ds(si, 1)],
                                         rows.at[2, pl.ds(lane0 + r, 1)],
                                         sems[2])
                    return 0

                lax.fori_loop(0, remv, ibody, 0)

            for bi in range(2):
                @pl.when(bi < nf)
                def _prime(bi=bi):
                    pltpu.async_copy(xe.at[gidx(gs + bi)], rows.at[bi],
                                     sems[bi])

            def outer(q, _):
                for bi in range(2):
                    k = q * 2 + bi
                    g = gs + k

                    @pl.when(k < nf)
                    def _step(bi=bi, g=g, k=k):
                        pltpu.make_async_copy(xe.at[gidx(g)], rows.at[bi],
                                              sems[bi]).wait()
                        pltpu.sync_copy(rows.at[bi], acc.at[didx(g)],
                                        add=True)

                        @pl.when(k + 2 < nf)
                        def _next():
                            pltpu.async_copy(xe.at[gidx(g + 2)],
                                             rows.at[bi], sems[bi])
                return 0

            lax.fori_loop(0, (nf + 1) // 2, outer, 0)

            @pl.when(rem > 0)
            def _drain_tail():
                def dbody(r, _):
                    pltpu.make_async_copy(xe.at[pl.ds(0, 1)],
                                          rows.at[2, pl.ds(0, 1)],
                                          sems[2]).wait()
                    return 0

                lax.fori_loop(0, remv * L, dbody, 0)
                pltpu.sync_copy(rows.at[2], acc.at[didx(pcg)], add=True)
            plsc.subcore_barrier()
            obase = dlo + h * NQ
            pltpu.async_copy(acc.at[pl.ds(s * DR, DR)],
                             out.at[j, pl.ds(obase + s * DR, DR)], sems[0])

            @pl.when(s == NS - 1)
            def _dump_tail():
                pltpu.async_copy(acc.at[pl.ds(NS * DR, DTAIL)],
                                 out.at[j, pl.ds(obase + NS * DR, DTAIL)],
                                 sems[0])

            return 0

        lax.fori_loop(0, NB, region, 0)

        pltpu.make_async_copy(acc.at[pl.ds(0, DR)], out.at[0, pl.ds(0, DR)],
                              sems[0]).wait()

        @pl.when(s == NS - 1)
        def _final_tail():
            pltpu.make_async_copy(acc.at[pl.ds(0, DTAIL)],
                                  out.at[0, pl.ds(0, DTAIL)], sems[0]).wait()

    return agg


def _make_dense(N, D, R, DE, BLK, emit_ext):
    def body(acc_ref, x_ref, w_ref, root_ref, b_ref, o_ref):
        x = x_ref[:, :D]
        y = jnp.dot(x, root_ref[...], preferred_element_type=jnp.float32)
        y = y + b_ref[...]
        for r in range(R):
            a = acc_ref[r, :, :D]
            cnt = acc_ref[r, :, D:D + 1]
            inv = 1.0 / jnp.maximum(cnt, 1.0)
            y = y + jnp.dot(a * inv, w_ref[r],
                            preferred_element_type=jnp.float32)
        if emit_ext:
            y = jnp.maximum(y, 0.0)
            col = lax.broadcasted_iota(jnp.int32, (BLK, DE - D), 1)
            tail = jnp.where(col == 0, 1.0, 0.0).astype(jnp.float32)
            o_ref[...] = jnp.concatenate([y, tail], axis=1)
        else:
            o_ref[...] = y

    out_w = DE if emit_ext else D
    return pl.pallas_call(
        body,
        grid=(N // BLK,),
        in_specs=[
            pl.BlockSpec((R, BLK, DE), lambda i: (0, i, 0)),
            pl.BlockSpec((BLK, DE), lambda i: (i, 0)),
            pl.BlockSpec((R, D, D), lambda i: (0, 0, 0)),
            pl.BlockSpec((D, D), lambda i: (0, 0)),
            pl.BlockSpec((D,), lambda i: (0,)),
        ],
        out_specs=pl.BlockSpec((BLK, out_w), lambda i: (i, 0)),
        out_shape=jax.ShapeDtypeStruct((N, out_w), jnp.float32),
    )


def kernel(edge_index, edge_type, emb, W1, root1, b1, W2, root2, b2):
    N, D = emb.shape
    E = edge_index.shape[1]
    R = W1.shape[0]
    DE = 136

    esrc = edge_index[0]
    edst = edge_index[1]

    bucket = _make_buckets(N, E, R)
    agg = _make_agg(N, E, D, R, DE)
    dense1 = _make_dense(N, D, R, DE, 1000, True)
    dense2 = _make_dense(N, D, R, DE, 1000, False)

    pad = jnp.zeros((N, DE - D - 1), jnp.float32)
    ones = jnp.ones((N, 1), jnp.float32)
    xe0 = jnp.concatenate([emb, ones, pad], axis=1)

    zin = jnp.zeros((128, DE), jnp.float32)
    bko, bcnt = bucket(esrc, edst, edge_type)
    acc1 = agg(xe0, zin, bko, bcnt)
    xe1 = dense1(acc1, xe0, W1, root1, b1)
    acc2 = agg(xe1, zin, bko, bcnt)
    y = dense2(acc2, xe1, W2, root2, b2)
    return y
